# Initial kernel scaffold; baseline (speedup 1.0000x reference)
import jax
import jax.numpy as jnp
from jax.experimental import pallas as pl


def kernel(x, pos_table):
    raise NotImplementedError("write your pallas kernel here")


# speedup vs baseline: 3.2906x; 3.2906x over previous
import jax
import jax.numpy as jnp
from jax.experimental import pallas as pl


def _add_pos_kernel(x_ref, pos_ref, out_ref):
    out_ref[...] = x_ref[...] + pos_ref[...][None, :, :]


def kernel(x, pos_table):
    batch, seq, d_model = x.shape
    bs = 512
    grid = (seq // bs,)
    return pl.pallas_call(
        _add_pos_kernel,
        grid=grid,
        in_specs=[
            pl.BlockSpec((batch, bs, d_model), lambda i: (0, i, 0)),
            pl.BlockSpec((bs, d_model), lambda i: (i, 0)),
        ],
        out_specs=pl.BlockSpec((batch, bs, d_model), lambda i: (0, i, 0)),
        out_shape=jax.ShapeDtypeStruct((batch, seq, d_model), x.dtype),
    )(x, pos_table[:seq])
